# baseline (speedup 1.0000x reference)
import functools

import jax
import jax.numpy as jnp
from jax import lax
from jax.experimental import pallas as pl
from jax.experimental.pallas import tpu as pltpu
from jax.experimental.pallas import tpu_sc as plsc

VOCAB = 1000
D = 128
L = 8
BATCH = 16384
B = BATCH * L

_info = plsc.get_sparse_core_info()
NC, NS, NLANES = _info.num_cores, _info.num_subcores, _info.num_lanes
NW = NC * NS
BPW = B // NW
R = 128
NCHUNK = BPW // R
W = D // 2
NK = D // (2 * NLANES)


def _body(tokw_hbm, idx_hbm, pos_hbm, out_hbm,
          tok_shw, idx_v, pos_v, in0, in1, ob0, ob1,
          g0, g1, o0, o1):
    sid = lax.axis_index("s")
    wid = sid * NC + lax.axis_index("c")
    base = wid * BPW

    @pl.when(sid == 0)
    def _stage_table():
        pltpu.sync_copy(tokw_hbm, tok_shw)

    pltpu.sync_copy(idx_hbm.at[pl.ds(base, BPW)], idx_v)
    pltpu.sync_copy(pos_hbm, pos_v)
    plsc.subcore_barrier()

    ins = (in0, in1)
    outs = (ob0, ob1)
    gsems = (g0, g1)
    osems = (o0, o1)
    himask = jnp.full((NLANES,), -65536, jnp.int32)

    def start_gather(g, bi):
        return pltpu.async_copy(
            tok_shw.at[idx_v.at[pl.ds(g * R, R)]], ins[bi], gsems[bi])

    def process(inb, outb):
        def k_body(k, _):
            c0 = k * 32
            plo = [pos_v[l, pl.ds(c0, NLANES)] for l in range(L)]
            phi = [pos_v[l, pl.ds(c0 + NLANES, NLANES)] for l in range(L)]

            def grp_body(grp, _):
                row0 = grp * L
                for l in range(L):
                    w = inb[row0 + l, pl.ds(NLANES * k, NLANES)]
                    a = lax.bitcast_convert_type(w << 16, jnp.float32)
                    b = lax.bitcast_convert_type(w & himask, jnp.float32)
                    outb[row0 + l, pl.ds(c0, NLANES)] = a + plo[l]
                    outb[row0 + l, pl.ds(c0 + NLANES, NLANES)] = b + phi[l]
                return 0

            lax.fori_loop(0, R // L, grp_body, 0)
            return 0

        lax.fori_loop(0, NK, k_body, 0)

    pending = [None, None]
    gcp = [None, None]
    gcp[0] = start_gather(0, 0)
    for g in range(NCHUNK):
        bi = g & 1
        gcp[bi].wait()
        if g + 1 < NCHUNK:
            gcp[1 - bi] = start_gather(g + 1, 1 - bi)
        if pending[bi] is not None:
            pending[bi].wait()
        process(ins[bi], outs[bi])
        pending[bi] = pltpu.async_copy(
            outs[bi], out_hbm.at[pl.ds(base + g * R, R)], osems[bi])
    pending[0].wait()
    pending[1].wait()


@functools.partial(jax.jit, static_argnums=())
def kernel(X, token_table, pos_table):
    idx = X.reshape(B)
    tb = token_table.astype(jnp.bfloat16).reshape(VOCAB, NK, 2, NLANES)
    tb = jnp.swapaxes(tb, 2, 3)
    tokw = lax.bitcast_convert_type(tb, jnp.int32).reshape(VOCAB, W)
    tokw = jnp.concatenate([tokw, tokw], axis=1)
    mesh = plsc.VectorSubcoreMesh(core_axis_name="c", subcore_axis_name="s")
    out = pl.kernel(
        _body,
        mesh=mesh,
        out_type=jax.ShapeDtypeStruct((B, D), jnp.float32),
        scratch_types=[
            pltpu.VMEM_SHARED((VOCAB, 2 * W), jnp.int32),
            pltpu.VMEM((BPW,), jnp.int32),
            pltpu.VMEM((L, D), jnp.float32),
            pltpu.VMEM((R, 2 * W), jnp.int32),
            pltpu.VMEM((R, 2 * W), jnp.int32),
            pltpu.VMEM((R, D), jnp.float32),
            pltpu.VMEM((R, D), jnp.float32),
            pltpu.SemaphoreType.DMA,
            pltpu.SemaphoreType.DMA,
            pltpu.SemaphoreType.DMA,
            pltpu.SemaphoreType.DMA,
        ],
    )(tokw, idx, pos_table)
    return out.reshape(BATCH, L, D)


# speedup vs baseline: 1.1002x; 1.1002x over previous
import functools

import jax
import jax.numpy as jnp
from jax import lax
from jax.experimental import pallas as pl
from jax.experimental.pallas import tpu as pltpu
from jax.experimental.pallas import tpu_sc as plsc

VOCAB = 1000
D = 128
L = 8
BATCH = 16384
B = BATCH * L

_info = plsc.get_sparse_core_info()
NC, NS, NLANES = _info.num_cores, _info.num_subcores, _info.num_lanes
NW = NC * NS
BPW = B // NW
R = 128
NCHUNK = BPW // R
NPAIR = NCHUNK // 2
W = D // 2
NK = D // (2 * NLANES)


def _body(tokw_hbm, idx_hbm, pos_hbm, out_hbm,
          tok_shw, idx_v, pos_v, in0, in1, ob0, ob1,
          g0, g1, o0, o1):
    sid = lax.axis_index("s")
    wid = sid * NC + lax.axis_index("c")
    base = wid * BPW

    @pl.when(sid == 0)
    def _stage_table():
        pltpu.sync_copy(tokw_hbm, tok_shw)

    pltpu.sync_copy(idx_hbm.at[pl.ds(base, BPW)], idx_v)
    pltpu.sync_copy(pos_hbm, pos_v)
    plsc.subcore_barrier()

    ins = (in0, in1)
    outs = (ob0, ob1)
    gsems = (g0, g1)
    osems = (o0, o1)
    himask = jnp.full((NLANES,), -65536, jnp.int32)

    def start_gather(off, bi):
        return pltpu.async_copy(
            tok_shw.at[idx_v.at[pl.ds(off, R)]], ins[bi], gsems[bi])

    def wait_gather(bi):
        pltpu.make_async_copy(
            tok_shw.at[idx_v.at[pl.ds(0, R)]], ins[bi], gsems[bi]).wait()

    def start_store(off, bi):
        return pltpu.async_copy(
            outs[bi], out_hbm.at[pl.ds(base + off, R)], osems[bi])

    def wait_store(bi):
        pltpu.make_async_copy(
            outs[bi], out_hbm.at[pl.ds(base, R)], osems[bi]).wait()

    def process(inb, outb):
        for k in range(NK):
            c0 = 32 * k
            plo = [pos_v[l, pl.ds(c0, NLANES)] for l in range(L)]
            phi = [pos_v[l, pl.ds(c0 + NLANES, NLANES)] for l in range(L)]

            def grp_body(grp, _):
                row0 = grp * L
                for l in range(L):
                    w = inb[row0 + l, pl.ds(NLANES * k, NLANES)]
                    a = lax.bitcast_convert_type(w << 16, jnp.float32)
                    b = lax.bitcast_convert_type(w & himask, jnp.float32)
                    outb[row0 + l, pl.ds(c0, NLANES)] = a + plo[l]
                    outb[row0 + l, pl.ds(c0 + NLANES, NLANES)] = b + phi[l]
                return 0

            lax.fori_loop(0, R // L, grp_body, 0)

    start_gather(0, 0)
    wait_gather(0)
    start_gather(R, 1)
    process(in0, ob0)
    start_store(0, 0)
    wait_gather(1)
    start_gather(2 * R, 0)
    process(in1, ob1)
    start_store(R, 1)

    def pair_body(p, _):
        ks = 2 * p * R
        wait_gather(0)
        start_gather(ks + R, 1)
        wait_store(0)
        process(in0, ob0)
        start_store(ks, 0)
        wait_gather(1)
        nxt = jnp.minimum(ks + 2 * R, (NCHUNK - 1) * R)
        start_gather(nxt, 0)
        wait_store(1)
        process(in1, ob1)
        start_store(ks + R, 1)
        return 0

    lax.fori_loop(1, NPAIR, pair_body, 0)

    wait_gather(0)
    wait_store(0)
    wait_store(1)


@functools.partial(jax.jit, static_argnums=())
def kernel(X, token_table, pos_table):
    idx = X.reshape(B)
    tb = token_table.astype(jnp.bfloat16).reshape(VOCAB, NK, 2, NLANES)
    tb = jnp.swapaxes(tb, 2, 3)
    tokw = lax.bitcast_convert_type(tb, jnp.int32).reshape(VOCAB, W)
    mesh = plsc.VectorSubcoreMesh(core_axis_name="c", subcore_axis_name="s")
    out = pl.kernel(
        _body,
        mesh=mesh,
        out_type=jax.ShapeDtypeStruct((B, D), jnp.float32),
        compiler_params=pltpu.CompilerParams(use_tc_tiling_on_sc=False),
        scratch_types=[
            pltpu.VMEM_SHARED((VOCAB, W), jnp.int32),
            pltpu.VMEM((BPW,), jnp.int32),
            pltpu.VMEM((L, D), jnp.float32),
            pltpu.VMEM((R, W), jnp.int32),
            pltpu.VMEM((R, W), jnp.int32),
            pltpu.VMEM((R, D), jnp.float32),
            pltpu.VMEM((R, D), jnp.float32),
            pltpu.SemaphoreType.DMA,
            pltpu.SemaphoreType.DMA,
            pltpu.SemaphoreType.DMA,
            pltpu.SemaphoreType.DMA,
        ],
    )(tokw, idx, pos_table)
    return out.reshape(BATCH, L, D)


# speedup vs baseline: 1.8887x; 1.7166x over previous
import functools

import jax
import jax.numpy as jnp
from jax import lax
from jax.experimental import pallas as pl
from jax.experimental.pallas import tpu as pltpu
from jax.experimental.pallas import tpu_sc as plsc

VOCAB = 1000
D = 128
L = 8
BATCH = 16384
B = BATCH * L

_info = plsc.get_sparse_core_info()
NC, NS, NLANES = _info.num_cores, _info.num_subcores, _info.num_lanes
NW = NC * NS
BPW = B // NW
R = 256
NCHUNK = BPW // R


def _body(tok_hbm, idx_hbm, pos_hbm, out_hbm, tok_sh, idx_v, buf0, buf1,
          pos_v, gsem0, gsem1, osem0, osem1):
    sid = lax.axis_index("s")
    wid = sid * NC + lax.axis_index("c")
    base = wid * BPW

    @pl.when(sid == 0)
    def _stage_table():
        pltpu.sync_copy(tok_hbm, tok_sh)

    pltpu.sync_copy(idx_hbm.at[pl.ds(base, BPW)], idx_v)
    pltpu.sync_copy(pos_hbm, pos_v)
    plsc.subcore_barrier()

    bufs = (buf0, buf1)
    gsems = (gsem0, gsem1)
    osems = (osem0, osem1)

    def start_gather(g, b):
        return pltpu.async_copy(
            tok_sh.at[idx_v.at[pl.ds(g * R, R)]], bufs[b], gsems[b])

    def add_pos(buf):
        def j_body(j, _):
            sl = pl.ds(j * NLANES, NLANES)
            prow = [pos_v[l, sl] for l in range(L)]

            def grp_body(grp, _):
                row0 = grp * L
                for l in range(L):
                    buf[row0 + l, sl] = buf[row0 + l, sl] + prow[l]
                return 0

            lax.fori_loop(0, R // L, grp_body, 0)
            return 0

        lax.fori_loop(0, D // NLANES, j_body, 0)

    gcp = [None, None]
    scp = [None, None]
    gcp[0] = start_gather(0, 0)
    for g in range(NCHUNK):
        b = g & 1
        gcp[b].wait()
        if g + 1 < NCHUNK:
            if scp[1 - b] is not None:
                scp[1 - b].wait()
            gcp[1 - b] = start_gather(g + 1, 1 - b)
        add_pos(bufs[b])
        scp[b] = pltpu.async_copy(
            bufs[b], out_hbm.at[pl.ds(base + g * R, R)], osems[b])
    scp[0].wait()
    scp[1].wait()


@functools.partial(jax.jit, static_argnums=())
def kernel(X, token_table, pos_table):
    idx = X.reshape(B)
    mesh = plsc.VectorSubcoreMesh(core_axis_name="c", subcore_axis_name="s")
    out = pl.kernel(
        _body,
        mesh=mesh,
        out_type=jax.ShapeDtypeStruct((B, D), jnp.float32),
        scratch_types=[
            pltpu.VMEM_SHARED((VOCAB, D), jnp.float32),
            pltpu.VMEM((BPW,), jnp.int32),
            pltpu.VMEM((R, D), jnp.float32),
            pltpu.VMEM((R, D), jnp.float32),
            pltpu.VMEM((L, D), jnp.float32),
            pltpu.SemaphoreType.DMA,
            pltpu.SemaphoreType.DMA,
            pltpu.SemaphoreType.DMA,
            pltpu.SemaphoreType.DMA,
        ],
    )(token_table, idx, pos_table)
    return out.reshape(BATCH, L, D)


# speedup vs baseline: 1.8890x; 1.0002x over previous
import functools

import jax
import jax.numpy as jnp
from jax import lax
from jax.experimental import pallas as pl
from jax.experimental.pallas import tpu as pltpu
from jax.experimental.pallas import tpu_sc as plsc

VOCAB = 1000
D = 128
L = 8
BATCH = 16384
B = BATCH * L

_info = plsc.get_sparse_core_info()
NC, NS, NLANES = _info.num_cores, _info.num_subcores, _info.num_lanes
NW = NC * NS
BPW = B // NW
R = 256
NCHUNK = BPW // R


def _body(tok_hbm, idx_hbm, pos_hbm, out_hbm, tok_sh, idx_v, buf0, buf1,
          pos_v, gsem0, gsem1, osem0, osem1):
    sid = lax.axis_index("s")
    wid = sid * NC + lax.axis_index("c")
    base = wid * BPW

    @pl.when(sid == 0)
    def _stage_table():
        pltpu.sync_copy(tok_hbm, tok_sh)

    pltpu.sync_copy(idx_hbm.at[pl.ds(base, BPW)], idx_v)
    pltpu.sync_copy(pos_hbm, pos_v)
    plsc.subcore_barrier()

    bufs = (buf0, buf1)
    gsems = (gsem0, gsem1)
    osems = (osem0, osem1)

    def start_gather(g, b):
        return pltpu.async_copy(
            tok_sh.at[idx_v.at[pl.ds(g * R, R)]], bufs[b], gsems[b])

    def add_pos(buf):
        def j_body(j, _):
            sl = pl.ds(j * NLANES, NLANES)
            prow = [pos_v[l, sl] for l in range(L)]

            def grp_body(grp, _):
                row0 = grp * L
                for l in range(L):
                    buf[row0 + l, sl] = buf[row0 + l, sl] + prow[l]
                return 0

            lax.fori_loop(0, R // L, grp_body, 0)
            return 0

        lax.fori_loop(0, D // NLANES, j_body, 0)

    gcp = [None, None]
    scp = [None, None]
    gcp[0] = start_gather(0, 0)
    for g in range(NCHUNK):
        b = g & 1
        gcp[b].wait()
        if g + 1 < NCHUNK:
            if scp[1 - b] is not None:
                scp[1 - b].wait()
            gcp[1 - b] = start_gather(g + 1, 1 - b)
        add_pos(bufs[b])
        scp[b] = pltpu.async_copy(
            bufs[b], out_hbm.at[pl.ds(base + g * R, R)], osems[b])
    scp[0].wait()
    scp[1].wait()


@functools.partial(jax.jit, static_argnums=())
def kernel(X, token_table, pos_table):
    idx = X.reshape(B)
    mesh = plsc.VectorSubcoreMesh(core_axis_name="c", subcore_axis_name="s")
    out = pl.kernel(
        _body,
        mesh=mesh,
        out_type=jax.ShapeDtypeStruct((B, D), jnp.float32),
        compiler_params=pltpu.CompilerParams(use_tc_tiling_on_sc=False),
        scratch_types=[
            pltpu.VMEM_SHARED((VOCAB, D), jnp.float32),
            pltpu.VMEM((BPW,), jnp.int32),
            pltpu.VMEM((R, D), jnp.float32),
            pltpu.VMEM((R, D), jnp.float32),
            pltpu.VMEM((L, D), jnp.float32),
            pltpu.SemaphoreType.DMA,
            pltpu.SemaphoreType.DMA,
            pltpu.SemaphoreType.DMA,
            pltpu.SemaphoreType.DMA,
        ],
    )(token_table, idx, pos_table)
    return out.reshape(BATCH, L, D)
